# baseline (speedup 1.0000x reference)
import functools

import jax
import jax.numpy as jnp
from jax import lax
from jax.experimental import pallas as pl
from jax.experimental.pallas import tpu as pltpu
from jax.experimental.pallas import tpu_sc as plsc

N = 10000
E = 320000
D = 128
ED = 16
G = 512
BN_EPS = 128.0
SLOPE = 0.01

NC = 2
NS = 16
NW = NC * NS
EPT = E // NW
C = 80
CH = EPT // C
GRP = 25
GCH = CH // GRP

NPT = 624
NTAIL = N - NPT * NS
PC = 104
PCH = NPT // PC
GPT = G // NS

_mesh = plsc.VectorSubcoreMesh(core_axis_name="c", subcore_axis_name="s")


def _lift_body(ea, w, b, o):
    o[...] = jnp.dot(ea[...], w[...],
                     preferred_element_type=jnp.float32) + b[...]


def _lift(edge_attr, We, be):
    BE = 8000
    grid = (E // BE,)
    return pl.pallas_call(
        _lift_body,
        grid=grid,
        in_specs=[pl.BlockSpec((BE, ED), lambda i: (i, 0)),
                  pl.BlockSpec((ED, D), lambda i: (0, 0)),
                  pl.BlockSpec((1, D), lambda i: (0, 0))],
        out_specs=pl.BlockSpec((BE, D), lambda i: (i, 0)),
        out_shape=jax.ShapeDtypeStruct((E, D), jnp.float32),
    )(edge_attr, We, be[None, :])


def _gine_body(x_hbm, z_hbm, e_hbm, src_hbm, dst_hbm, out_hbm,
               aggr_sh, src_v, dst_v, ebuf, xbuf,
               es0, es1, xs0, xs1, ss0, ss1):
    esem = (es0, es1)
    xsem = (xs0, xs1)
    ssem = (ss0, ss1)
    c = lax.axis_index("c")
    s = lax.axis_index("s")
    wid = s * NC + c
    row0 = pl.multiple_of(s * NPT, 8)

    @pl.when(c == 0)
    def _():
        pltpu.sync_copy(x_hbm.at[pl.ds(row0, NPT)], aggr_sh.at[pl.ds(row0, NPT)])

    @pl.when(c != 0)
    def _():
        pltpu.sync_copy(z_hbm.at[pl.ds(row0, NPT)], aggr_sh.at[pl.ds(row0, NPT)])

    @pl.when(jnp.logical_and(c == 0, s == NS - 1))
    def _():
        pltpu.sync_copy(x_hbm.at[pl.ds(NPT * NS, NTAIL)],
                        aggr_sh.at[pl.ds(NPT * NS, NTAIL)])

    @pl.when(jnp.logical_and(c != 0, s == NS - 1))
    def _():
        pltpu.sync_copy(z_hbm.at[pl.ds(NPT * NS, NTAIL)],
                        aggr_sh.at[pl.ds(NPT * NS, NTAIL)])

    plsc.subcore_barrier()

    ebase = wid * EPT

    def group(gi, carry):
        pltpu.sync_copy(src_hbm.at[wid, gi], src_v)
        pltpu.sync_copy(dst_hbm.at[wid, gi], dst_v)
        g0 = gi * GRP
        e_descs = [None] * GRP
        x_descs = [None] * GRP
        s_descs = [None] * GRP

        def start_chunk(q):
            b = q % 2
            e0 = pl.multiple_of(ebase + (g0 + q) * C, 8)
            e_descs[q] = pltpu.async_copy(
                e_hbm.at[pl.ds(e0, C)], ebuf.at[b], esem[b])
            x_descs[q] = pltpu.async_copy(
                x_hbm.at[src_v.at[q]], xbuf.at[b], xsem[b])

        start_chunk(0)
        for q in range(GRP):
            b = q % 2
            if q + 1 < GRP:
                start_chunk(q + 1)
            e_descs[q].wait()
            x_descs[q].wait()

            @plsc.parallel_loop(0, C, 1, unroll=4)
            def _(r):
                for k in range(D // 16):
                    sl_ = pl.ds(k * 16, 16)
                    v = xbuf[b, r, sl_] + ebuf[b, r, sl_]
                    ebuf[b, r, sl_] = jnp.maximum(v, 0.0)
            pltpu.sync_copy(ebuf.at[b], aggr_sh.at[dst_v.at[q]], add=True)
        return carry

    lax.fori_loop(0, GCH, group, 0)
    plsc.subcore_barrier()
    pltpu.sync_copy(aggr_sh.at[pl.ds(row0, NPT)],
                    out_hbm.at[c, pl.ds(row0, NPT)])

    @pl.when(s == NS - 1)
    def _():
        pltpu.sync_copy(aggr_sh.at[pl.ds(NPT * NS, NTAIL)],
                        out_hbm.at[c, pl.ds(NPT * NS, NTAIL)])


_gine_sc = functools.partial(
    pl.kernel,
    out_type=jax.ShapeDtypeStruct((NC, N, D), jnp.float32),
    mesh=_mesh,
    scratch_types=[
        pltpu.VMEM_SHARED((N, D), jnp.float32),
        pltpu.VMEM((GRP, C), jnp.int32),
        pltpu.VMEM((GRP, C), jnp.int32),
        pltpu.VMEM((2, C, D), jnp.float32),
        pltpu.VMEM((2, C, D), jnp.float32),
        pltpu.SemaphoreType.DMA,
        pltpu.SemaphoreType.DMA,
        pltpu.SemaphoreType.DMA,
        pltpu.SemaphoreType.DMA,
        pltpu.SemaphoreType.DMA,
        pltpu.SemaphoreType.DMA,
    ],
)(_gine_body)


def _dense_body(a0, a1, w, b, g, bt, o):
    t = a0[...] + a1[...]
    hl = jnp.dot(t, w[...], preferred_element_type=jnp.float32) + b[...]
    mu = jnp.mean(hl, axis=0, keepdims=True)
    var = jnp.mean((hl - mu) ** 2, axis=0, keepdims=True)
    hn = (hl - mu) / jnp.sqrt(var + BN_EPS) * g[...] + bt[...]
    o[...] = jnp.where(hn > 0, hn, SLOPE * hn)


def _dense(agg, w, b, g, bt):
    return pl.pallas_call(
        _dense_body,
        out_shape=jax.ShapeDtypeStruct((N, D), jnp.float32),
    )(agg[0], agg[1], w, b[None, :], g[None, :], bt[None, :])


def _pool_body(h_hbm, z_hbm, b_hbm, out_hbm, pool_sh, bidx_v, hbuf):
    c = lax.axis_index("c")
    s = lax.axis_index("s")
    row0 = pl.multiple_of(s * NPT, 8)

    @pl.when(c == 0)
    def _():
        pltpu.sync_copy(z_hbm.at[pl.ds(s * GPT, GPT)],
                        pool_sh.at[pl.ds(s * GPT, GPT)])
        for t in range(PCH):
            pltpu.sync_copy(b_hbm.at[pl.ds(row0 + t * PC, PC)], bidx_v.at[t])

    @pl.when(jnp.logical_and(c == 0, s == NS - 1))
    def _():
        pltpu.sync_copy(b_hbm.at[pl.ds(NPT * NS, NTAIL)],
                        bidx_v.at[PCH, pl.ds(0, NTAIL)])

    plsc.subcore_barrier()

    @pl.when(c == 0)
    def _():
        for t in range(PCH):
            pltpu.sync_copy(h_hbm.at[pl.ds(row0 + t * PC, PC)],
                            hbuf.at[pl.ds(0, PC)])
            pltpu.sync_copy(hbuf.at[pl.ds(0, PC)],
                            pool_sh.at[bidx_v.at[t]], add=True)

    @pl.when(jnp.logical_and(c == 0, s == NS - 1))
    def _():
        pltpu.sync_copy(h_hbm.at[pl.ds(NPT * NS, NTAIL)],
                        hbuf.at[pl.ds(0, NTAIL)])
        pltpu.sync_copy(hbuf.at[pl.ds(0, NTAIL)],
                        pool_sh.at[bidx_v.at[PCH, pl.ds(0, NTAIL)]], add=True)

    plsc.subcore_barrier()

    @pl.when(c == 0)
    def _():
        for t in range(PCH):
            pltpu.sync_copy(pool_sh.at[bidx_v.at[t]], hbuf.at[pl.ds(0, PC)])
            pltpu.sync_copy(hbuf.at[pl.ds(0, PC)],
                            out_hbm.at[pl.ds(row0 + t * PC, PC)])

    @pl.when(jnp.logical_and(c == 0, s == NS - 1))
    def _():
        pltpu.sync_copy(pool_sh.at[bidx_v.at[PCH, pl.ds(0, NTAIL)]],
                        hbuf.at[pl.ds(0, NTAIL)])
        pltpu.sync_copy(hbuf.at[pl.ds(0, NTAIL)],
                        out_hbm.at[pl.ds(NPT * NS, NTAIL)])


_pool_sc = functools.partial(
    pl.kernel,
    out_type=jax.ShapeDtypeStruct((N, D), jnp.float32),
    mesh=_mesh,
    scratch_types=[
        pltpu.VMEM_SHARED((G, D), jnp.float32),
        pltpu.VMEM((PCH + 1, PC), jnp.int32),
        pltpu.VMEM((PC, D), jnp.float32),
    ],
)(_pool_body)


def _final_body(h1, h2, h3, hp, a1, a2, a3, a4, bl1, wl2, bl2, o):
    z = (jnp.dot(h1[...], a1[...], preferred_element_type=jnp.float32)
         + jnp.dot(h2[...], a2[...], preferred_element_type=jnp.float32)
         + jnp.dot(h3[...], a3[...], preferred_element_type=jnp.float32)
         + jnp.dot(hp[...], a4[...], preferred_element_type=jnp.float32)
         + bl1[...])
    z = jnp.where(z > 0, z, SLOPE * z)
    o[...] = jnp.dot(z, wl2[...], preferred_element_type=jnp.float32) + bl2[...]


def _final(h1, h2, h3, hp, Wl1, bl1, Wl2, bl2):
    BR = 2000
    grid = (N // BR,)
    hspec = pl.BlockSpec((BR, D), lambda i: (i, 0))
    aspec = pl.BlockSpec((D, 4 * D), lambda i: (0, 0))
    a1, a2, a3, a4 = jnp.split(Wl1, 4, axis=0)
    return pl.pallas_call(
        _final_body,
        grid=grid,
        in_specs=[hspec, hspec, hspec, hspec,
                  aspec, aspec, aspec, aspec,
                  pl.BlockSpec((1, 4 * D), lambda i: (0, 0)),
                  pl.BlockSpec((4 * D, 1), lambda i: (0, 0)),
                  pl.BlockSpec((1, 1), lambda i: (0, 0))],
        out_specs=pl.BlockSpec((BR, 1), lambda i: (i, 0)),
        out_shape=jax.ShapeDtypeStruct((N, 1), jnp.float32),
    )(h1, h2, h3, hp, a1, a2, a3, a4, bl1[None, :], Wl2, bl2[None, :])


def kernel(x, edge_index, edge_attr, batch,
           We1, be1, W1, b1, g1, bt1,
           We2, be2, W2, b2, g2, bt2,
           We3, be3, W3, b3, g3, bt3,
           Wl1, bl1, Wl2, bl2):
    src = edge_index[0].reshape(NW, GCH, GRP, C)
    dst = edge_index[1].reshape(NW, GCH, GRP, C)
    zeros = jnp.zeros((N, D), jnp.float32)

    e1 = _lift(edge_attr, We1, be1)
    e2 = _lift(edge_attr, We2, be2)
    e3 = _lift(edge_attr, We3, be3)

    agg1 = _gine_sc(x, zeros, e1, src, dst)
    h1 = _dense(agg1, W1, b1, g1, bt1)
    agg2 = _gine_sc(h1, zeros, e2, src, dst)
    h2 = _dense(agg2, W2, b2, g2, bt2)
    agg3 = _gine_sc(h2, zeros, e3, src, dst)
    h3 = _dense(agg3, W3, b3, g3, bt3)

    hp = _pool_sc(h3, zeros, batch)
    return _final(h1, h2, h3, hp, Wl1, bl1, Wl2, bl2)


# speedup vs baseline: 1.0133x; 1.0133x over previous
import functools

import jax
import jax.numpy as jnp
from jax import lax
from jax.experimental import pallas as pl
from jax.experimental.pallas import tpu as pltpu
from jax.experimental.pallas import tpu_sc as plsc

N = 10000
E = 320000
D = 128
ED = 16
G = 512
BN_EPS = 128.0
SLOPE = 0.01

NC = 2
NS = 16
NW = NC * NS
EPT = E // NW
C = 80
CH = EPT // C
GRP = 25
GCH = CH // GRP

NPT = 624
NTAIL = N - NPT * NS
PC = 104
PCH = NPT // PC
GPT = G // NS

_mesh = plsc.VectorSubcoreMesh(core_axis_name="c", subcore_axis_name="s")


def _lift_body(ea, w, b, o):
    o[...] = jnp.dot(ea[...], w[...],
                     preferred_element_type=jnp.float32) + b[...]


def _lift(edge_attr, We, be):
    BE = 8000
    grid = (E // BE,)
    return pl.pallas_call(
        _lift_body,
        grid=grid,
        in_specs=[pl.BlockSpec((BE, ED), lambda i: (i, 0)),
                  pl.BlockSpec((ED, D), lambda i: (0, 0)),
                  pl.BlockSpec((1, D), lambda i: (0, 0))],
        out_specs=pl.BlockSpec((BE, D), lambda i: (i, 0)),
        out_shape=jax.ShapeDtypeStruct((E, D), jnp.float32),
    )(edge_attr, We, be[None, :])


def _gine_body(x_hbm, z_hbm, e_hbm, src_hbm, dst_hbm, out_hbm,
               aggr_sh, src_v, dst_v, ebuf, xbuf,
               es0, es1, xs0, xs1, ss0, ss1):
    esem = (es0, es1)
    xsem = (xs0, xs1)
    ssem = (ss0, ss1)
    c = lax.axis_index("c")
    s = lax.axis_index("s")
    wid = s * NC + c
    row0 = pl.multiple_of(s * NPT, 8)

    @pl.when(c == 0)
    def _():
        pltpu.sync_copy(x_hbm.at[pl.ds(row0, NPT)], aggr_sh.at[pl.ds(row0, NPT)])

    @pl.when(c != 0)
    def _():
        pltpu.sync_copy(z_hbm.at[pl.ds(row0, NPT)], aggr_sh.at[pl.ds(row0, NPT)])

    @pl.when(jnp.logical_and(c == 0, s == NS - 1))
    def _():
        pltpu.sync_copy(x_hbm.at[pl.ds(NPT * NS, NTAIL)],
                        aggr_sh.at[pl.ds(NPT * NS, NTAIL)])

    @pl.when(jnp.logical_and(c != 0, s == NS - 1))
    def _():
        pltpu.sync_copy(z_hbm.at[pl.ds(NPT * NS, NTAIL)],
                        aggr_sh.at[pl.ds(NPT * NS, NTAIL)])

    plsc.subcore_barrier()

    ebase = wid * EPT

    def group(gi, carry):
        pltpu.sync_copy(src_hbm.at[wid, gi], src_v)
        pltpu.sync_copy(dst_hbm.at[wid, gi], dst_v)
        g0 = gi * GRP
        e_descs = [None] * GRP
        x_descs = [None] * GRP
        s_descs = [None] * GRP

        def start_chunk(q):
            b = q % 2
            e0 = pl.multiple_of(ebase + (g0 + q) * C, 8)
            e_descs[q] = pltpu.async_copy(
                e_hbm.at[pl.ds(e0, C)], ebuf.at[b], esem[b])
            x_descs[q] = pltpu.async_copy(
                x_hbm.at[src_v.at[q]], xbuf.at[b], xsem[b])

        start_chunk(0)
        for q in range(GRP):
            b = q % 2
            if q + 1 < GRP:
                start_chunk(q + 1)
            e_descs[q].wait()
            x_descs[q].wait()

            def row(r, carry2):
                for k in range(D // 16):
                    sl_ = pl.ds(k * 16, 16)
                    v = xbuf[b, r, sl_] + ebuf[b, r, sl_]
                    ebuf[b, r, sl_] = jnp.maximum(v, 0.0)
                return carry2

            lax.fori_loop(0, C, row, 0)
            pltpu.sync_copy(ebuf.at[b], aggr_sh.at[dst_v.at[q]], add=True)
        return carry

    lax.fori_loop(0, GCH, group, 0)
    plsc.subcore_barrier()
    pltpu.sync_copy(aggr_sh.at[pl.ds(row0, NPT)],
                    out_hbm.at[c, pl.ds(row0, NPT)])

    @pl.when(s == NS - 1)
    def _():
        pltpu.sync_copy(aggr_sh.at[pl.ds(NPT * NS, NTAIL)],
                        out_hbm.at[c, pl.ds(NPT * NS, NTAIL)])


_gine_sc = functools.partial(
    pl.kernel,
    out_type=jax.ShapeDtypeStruct((NC, N, D), jnp.float32),
    mesh=_mesh,
    scratch_types=[
        pltpu.VMEM_SHARED((N, D), jnp.float32),
        pltpu.VMEM((GRP, C), jnp.int32),
        pltpu.VMEM((GRP, C), jnp.int32),
        pltpu.VMEM((2, C, D), jnp.float32),
        pltpu.VMEM((2, C, D), jnp.float32),
        pltpu.SemaphoreType.DMA,
        pltpu.SemaphoreType.DMA,
        pltpu.SemaphoreType.DMA,
        pltpu.SemaphoreType.DMA,
        pltpu.SemaphoreType.DMA,
        pltpu.SemaphoreType.DMA,
    ],
)(_gine_body)


def _dense_body(a0, a1, w, b, g, bt, o):
    t = a0[...] + a1[...]
    hl = jnp.dot(t, w[...], preferred_element_type=jnp.float32) + b[...]
    mu = jnp.mean(hl, axis=0, keepdims=True)
    var = jnp.mean((hl - mu) ** 2, axis=0, keepdims=True)
    hn = (hl - mu) / jnp.sqrt(var + BN_EPS) * g[...] + bt[...]
    o[...] = jnp.where(hn > 0, hn, SLOPE * hn)


def _dense(agg, w, b, g, bt):
    return pl.pallas_call(
        _dense_body,
        out_shape=jax.ShapeDtypeStruct((N, D), jnp.float32),
    )(agg[0], agg[1], w, b[None, :], g[None, :], bt[None, :])


def _pool_body(h_hbm, z_hbm, b_hbm, out_hbm, pool_sh, bidx_v, hbuf):
    c = lax.axis_index("c")
    s = lax.axis_index("s")
    row0 = pl.multiple_of(s * NPT, 8)

    @pl.when(c == 0)
    def _():
        pltpu.sync_copy(z_hbm.at[pl.ds(s * GPT, GPT)],
                        pool_sh.at[pl.ds(s * GPT, GPT)])
        for t in range(PCH):
            pltpu.sync_copy(b_hbm.at[pl.ds(row0 + t * PC, PC)], bidx_v.at[t])

    @pl.when(jnp.logical_and(c == 0, s == NS - 1))
    def _():
        pltpu.sync_copy(b_hbm.at[pl.ds(NPT * NS, NTAIL)],
                        bidx_v.at[PCH, pl.ds(0, NTAIL)])

    plsc.subcore_barrier()

    @pl.when(c == 0)
    def _():
        for t in range(PCH):
            pltpu.sync_copy(h_hbm.at[pl.ds(row0 + t * PC, PC)],
                            hbuf.at[pl.ds(0, PC)])
            pltpu.sync_copy(hbuf.at[pl.ds(0, PC)],
                            pool_sh.at[bidx_v.at[t]], add=True)

    @pl.when(jnp.logical_and(c == 0, s == NS - 1))
    def _():
        pltpu.sync_copy(h_hbm.at[pl.ds(NPT * NS, NTAIL)],
                        hbuf.at[pl.ds(0, NTAIL)])
        pltpu.sync_copy(hbuf.at[pl.ds(0, NTAIL)],
                        pool_sh.at[bidx_v.at[PCH, pl.ds(0, NTAIL)]], add=True)

    plsc.subcore_barrier()

    @pl.when(c == 0)
    def _():
        for t in range(PCH):
            pltpu.sync_copy(pool_sh.at[bidx_v.at[t]], hbuf.at[pl.ds(0, PC)])
            pltpu.sync_copy(hbuf.at[pl.ds(0, PC)],
                            out_hbm.at[pl.ds(row0 + t * PC, PC)])

    @pl.when(jnp.logical_and(c == 0, s == NS - 1))
    def _():
        pltpu.sync_copy(pool_sh.at[bidx_v.at[PCH, pl.ds(0, NTAIL)]],
                        hbuf.at[pl.ds(0, NTAIL)])
        pltpu.sync_copy(hbuf.at[pl.ds(0, NTAIL)],
                        out_hbm.at[pl.ds(NPT * NS, NTAIL)])


_pool_sc = functools.partial(
    pl.kernel,
    out_type=jax.ShapeDtypeStruct((N, D), jnp.float32),
    mesh=_mesh,
    scratch_types=[
        pltpu.VMEM_SHARED((G, D), jnp.float32),
        pltpu.VMEM((PCH + 1, PC), jnp.int32),
        pltpu.VMEM((PC, D), jnp.float32),
    ],
)(_pool_body)


def _final_body(h1, h2, h3, hp, a1, a2, a3, a4, bl1, wl2, bl2, o):
    z = (jnp.dot(h1[...], a1[...], preferred_element_type=jnp.float32)
         + jnp.dot(h2[...], a2[...], preferred_element_type=jnp.float32)
         + jnp.dot(h3[...], a3[...], preferred_element_type=jnp.float32)
         + jnp.dot(hp[...], a4[...], preferred_element_type=jnp.float32)
         + bl1[...])
    z = jnp.where(z > 0, z, SLOPE * z)
    o[...] = jnp.dot(z, wl2[...], preferred_element_type=jnp.float32) + bl2[...]


def _final(h1, h2, h3, hp, Wl1, bl1, Wl2, bl2):
    BR = 2000
    grid = (N // BR,)
    hspec = pl.BlockSpec((BR, D), lambda i: (i, 0))
    aspec = pl.BlockSpec((D, 4 * D), lambda i: (0, 0))
    a1, a2, a3, a4 = jnp.split(Wl1, 4, axis=0)
    return pl.pallas_call(
        _final_body,
        grid=grid,
        in_specs=[hspec, hspec, hspec, hspec,
                  aspec, aspec, aspec, aspec,
                  pl.BlockSpec((1, 4 * D), lambda i: (0, 0)),
                  pl.BlockSpec((4 * D, 1), lambda i: (0, 0)),
                  pl.BlockSpec((1, 1), lambda i: (0, 0))],
        out_specs=pl.BlockSpec((BR, 1), lambda i: (i, 0)),
        out_shape=jax.ShapeDtypeStruct((N, 1), jnp.float32),
    )(h1, h2, h3, hp, a1, a2, a3, a4, bl1[None, :], Wl2, bl2[None, :])


def kernel(x, edge_index, edge_attr, batch,
           We1, be1, W1, b1, g1, bt1,
           We2, be2, W2, b2, g2, bt2,
           We3, be3, W3, b3, g3, bt3,
           Wl1, bl1, Wl2, bl2):
    src = edge_index[0].reshape(NW, GCH, GRP, C)
    dst = edge_index[1].reshape(NW, GCH, GRP, C)
    zeros = jnp.zeros((N, D), jnp.float32)

    e1 = _lift(edge_attr, We1, be1)
    e2 = _lift(edge_attr, We2, be2)
    e3 = _lift(edge_attr, We3, be3)

    agg1 = _gine_sc(x, zeros, e1, src, dst)
    h1 = _dense(agg1, W1, b1, g1, bt1)
    agg2 = _gine_sc(h1, zeros, e2, src, dst)
    h2 = _dense(agg2, W2, b2, g2, bt2)
    agg3 = _gine_sc(h2, zeros, e3, src, dst)
    h3 = _dense(agg3, W3, b3, g3, bt3)

    hp = _pool_sc(h3, zeros, batch)
    return _final(h1, h2, h3, hp, Wl1, bl1, Wl2, bl2)


# speedup vs baseline: 1.0486x; 1.0348x over previous
import functools

import jax
import jax.numpy as jnp
from jax import lax
from jax.experimental import pallas as pl
from jax.experimental.pallas import tpu as pltpu
from jax.experimental.pallas import tpu_sc as plsc

N = 10000
E = 320000
D = 128
ED = 16
G = 512
BN_EPS = 128.0
SLOPE = 0.01

NC = 2
NS = 16
NW = NC * NS
EPT = E // NW
C = 80
CH = EPT // C
GRP = 25
GCH = CH // GRP

NPT = 624
NTAIL = N - NPT * NS
PC = 104
PCH = NPT // PC
GPT = G // NS

_mesh = plsc.VectorSubcoreMesh(core_axis_name="c", subcore_axis_name="s")


def _lift_body(ea, w1, b1, w2, b2, w3, b3, o1, o2, o3):
    a = ea[...]
    o1[...] = jnp.dot(a, w1[...], preferred_element_type=jnp.float32) + b1[...]
    o2[...] = jnp.dot(a, w2[...], preferred_element_type=jnp.float32) + b2[...]
    o3[...] = jnp.dot(a, w3[...], preferred_element_type=jnp.float32) + b3[...]


def _lift(edge_attr, We1, be1, We2, be2, We3, be3):
    BE = 8000
    grid = (E // BE,)
    wspec = pl.BlockSpec((ED, D), lambda i: (0, 0))
    bspec = pl.BlockSpec((1, D), lambda i: (0, 0))
    ospec = pl.BlockSpec((BE, D), lambda i: (i, 0))
    return pl.pallas_call(
        _lift_body,
        grid=grid,
        in_specs=[pl.BlockSpec((BE, ED), lambda i: (i, 0)),
                  wspec, bspec, wspec, bspec, wspec, bspec],
        out_specs=[ospec, ospec, ospec],
        out_shape=[jax.ShapeDtypeStruct((E, D), jnp.float32)] * 3,
    )(edge_attr, We1, be1[None, :], We2, be2[None, :], We3, be3[None, :])


def _gine_body(x_hbm, z_hbm, e_hbm, src_hbm, dst_hbm, out_hbm,
               aggr_sh, src_v, dst_v, ebuf, xbuf,
               es0, es1, xs0, xs1, ssems):
    esem = (es0, es1)
    xsem = (xs0, xs1)
    c = lax.axis_index("c")
    s = lax.axis_index("s")
    wid = s * NC + c
    row0 = pl.multiple_of(s * NPT, 8)

    @pl.when(c == 0)
    def _():
        pltpu.sync_copy(x_hbm.at[pl.ds(row0, NPT)], aggr_sh.at[pl.ds(row0, NPT)])

    @pl.when(c != 0)
    def _():
        pltpu.sync_copy(z_hbm.at[pl.ds(row0, NPT)], aggr_sh.at[pl.ds(row0, NPT)])

    @pl.when(jnp.logical_and(c == 0, s == NS - 1))
    def _():
        pltpu.sync_copy(x_hbm.at[pl.ds(NPT * NS, NTAIL)],
                        aggr_sh.at[pl.ds(NPT * NS, NTAIL)])

    @pl.when(jnp.logical_and(c != 0, s == NS - 1))
    def _():
        pltpu.sync_copy(z_hbm.at[pl.ds(NPT * NS, NTAIL)],
                        aggr_sh.at[pl.ds(NPT * NS, NTAIL)])

    plsc.subcore_barrier()

    ebase = wid * EPT

    def group(gi, carry):
        pltpu.sync_copy(src_hbm.at[wid, gi], src_v)
        pltpu.sync_copy(dst_hbm.at[wid, gi], dst_v)
        g0 = gi * GRP
        e_descs = [None] * GRP
        x_descs = [None] * GRP
        s_descs = [None] * GRP

        def start_chunk(q):
            b = q % 2
            e0 = pl.multiple_of(ebase + (g0 + q) * C, 8)
            e_descs[q] = pltpu.async_copy(
                e_hbm.at[pl.ds(e0, C)], ebuf.at[b], esem[b])
            x_descs[q] = pltpu.async_copy(
                x_hbm.at[src_v.at[q]], xbuf.at[b], xsem[b])

        start_chunk(0)
        for q in range(GRP):
            b = q % 2
            if q + 1 < GRP:
                if q >= 1:
                    s_descs[q - 1].wait()
                start_chunk(q + 1)
            e_descs[q].wait()
            x_descs[q].wait()

            def row(r, carry2):
                for k in range(D // 16):
                    sl_ = pl.ds(k * 16, 16)
                    v = xbuf[b, r, sl_] + ebuf[b, r, sl_]
                    ebuf[b, r, sl_] = jnp.maximum(v, 0.0)
                return carry2

            lax.fori_loop(0, C, row, 0)
            s_descs[q] = pltpu.async_copy(
                ebuf.at[b], aggr_sh.at[dst_v.at[q]], ssems.at[q], add=True)
        s_descs[GRP - 2].wait()
        s_descs[GRP - 1].wait()
        return carry

    lax.fori_loop(0, GCH, group, 0)
    plsc.subcore_barrier()
    pltpu.sync_copy(aggr_sh.at[pl.ds(row0, NPT)],
                    out_hbm.at[c, pl.ds(row0, NPT)])

    @pl.when(s == NS - 1)
    def _():
        pltpu.sync_copy(aggr_sh.at[pl.ds(NPT * NS, NTAIL)],
                        out_hbm.at[c, pl.ds(NPT * NS, NTAIL)])


_gine_sc = functools.partial(
    pl.kernel,
    out_type=jax.ShapeDtypeStruct((NC, N, D), jnp.float32),
    mesh=_mesh,
    scratch_types=[
        pltpu.VMEM_SHARED((N, D), jnp.float32),
        pltpu.VMEM((GRP, C), jnp.int32),
        pltpu.VMEM((GRP, C), jnp.int32),
        pltpu.VMEM((2, C, D), jnp.float32),
        pltpu.VMEM((2, C, D), jnp.float32),
        pltpu.SemaphoreType.DMA,
        pltpu.SemaphoreType.DMA,
        pltpu.SemaphoreType.DMA,
        pltpu.SemaphoreType.DMA,
        pltpu.SemaphoreType.DMA((GRP,)),
    ],
)(_gine_body)


def _dense_body(a0, a1, w, b, g, bt, o):
    t = a0[...] + a1[...]
    hl = jnp.dot(t, w[...], preferred_element_type=jnp.float32) + b[...]
    mu = jnp.mean(hl, axis=0, keepdims=True)
    var = jnp.mean((hl - mu) ** 2, axis=0, keepdims=True)
    hn = (hl - mu) / jnp.sqrt(var + BN_EPS) * g[...] + bt[...]
    o[...] = jnp.where(hn > 0, hn, SLOPE * hn)


def _dense(agg, w, b, g, bt):
    return pl.pallas_call(
        _dense_body,
        out_shape=jax.ShapeDtypeStruct((N, D), jnp.float32),
    )(agg[0], agg[1], w, b[None, :], g[None, :], bt[None, :])


def _pool_body(h_hbm, z_hbm, b_hbm, out_hbm, pool_sh, bidx_v, hbuf):
    c = lax.axis_index("c")
    s = lax.axis_index("s")
    row0 = pl.multiple_of(s * NPT, 8)

    @pl.when(c == 0)
    def _():
        pltpu.sync_copy(z_hbm.at[pl.ds(s * GPT, GPT)],
                        pool_sh.at[pl.ds(s * GPT, GPT)])
        for t in range(PCH):
            pltpu.sync_copy(b_hbm.at[pl.ds(row0 + t * PC, PC)], bidx_v.at[t])

    @pl.when(jnp.logical_and(c == 0, s == NS - 1))
    def _():
        pltpu.sync_copy(b_hbm.at[pl.ds(NPT * NS, NTAIL)],
                        bidx_v.at[PCH, pl.ds(0, NTAIL)])

    plsc.subcore_barrier()

    @pl.when(c == 0)
    def _():
        for t in range(PCH):
            pltpu.sync_copy(h_hbm.at[pl.ds(row0 + t * PC, PC)],
                            hbuf.at[pl.ds(0, PC)])
            pltpu.sync_copy(hbuf.at[pl.ds(0, PC)],
                            pool_sh.at[bidx_v.at[t]], add=True)

    @pl.when(jnp.logical_and(c == 0, s == NS - 1))
    def _():
        pltpu.sync_copy(h_hbm.at[pl.ds(NPT * NS, NTAIL)],
                        hbuf.at[pl.ds(0, NTAIL)])
        pltpu.sync_copy(hbuf.at[pl.ds(0, NTAIL)],
                        pool_sh.at[bidx_v.at[PCH, pl.ds(0, NTAIL)]], add=True)

    plsc.subcore_barrier()

    @pl.when(c == 0)
    def _():
        for t in range(PCH):
            pltpu.sync_copy(pool_sh.at[bidx_v.at[t]], hbuf.at[pl.ds(0, PC)])
            pltpu.sync_copy(hbuf.at[pl.ds(0, PC)],
                            out_hbm.at[pl.ds(row0 + t * PC, PC)])

    @pl.when(jnp.logical_and(c == 0, s == NS - 1))
    def _():
        pltpu.sync_copy(pool_sh.at[bidx_v.at[PCH, pl.ds(0, NTAIL)]],
                        hbuf.at[pl.ds(0, NTAIL)])
        pltpu.sync_copy(hbuf.at[pl.ds(0, NTAIL)],
                        out_hbm.at[pl.ds(NPT * NS, NTAIL)])


_pool_sc = functools.partial(
    pl.kernel,
    out_type=jax.ShapeDtypeStruct((N, D), jnp.float32),
    mesh=_mesh,
    scratch_types=[
        pltpu.VMEM_SHARED((G, D), jnp.float32),
        pltpu.VMEM((PCH + 1, PC), jnp.int32),
        pltpu.VMEM((PC, D), jnp.float32),
    ],
)(_pool_body)


def _final_body(h1, h2, h3, hp, a1, a2, a3, a4, bl1, wl2, bl2, o):
    z = (jnp.dot(h1[...], a1[...], preferred_element_type=jnp.float32)
         + jnp.dot(h2[...], a2[...], preferred_element_type=jnp.float32)
         + jnp.dot(h3[...], a3[...], preferred_element_type=jnp.float32)
         + jnp.dot(hp[...], a4[...], preferred_element_type=jnp.float32)
         + bl1[...])
    z = jnp.where(z > 0, z, SLOPE * z)
    o[...] = jnp.dot(z, wl2[...], preferred_element_type=jnp.float32) + bl2[...]


def _final(h1, h2, h3, hp, Wl1, bl1, Wl2, bl2):
    BR = 2000
    grid = (N // BR,)
    hspec = pl.BlockSpec((BR, D), lambda i: (i, 0))
    aspec = pl.BlockSpec((D, 4 * D), lambda i: (0, 0))
    a1, a2, a3, a4 = jnp.split(Wl1, 4, axis=0)
    return pl.pallas_call(
        _final_body,
        grid=grid,
        in_specs=[hspec, hspec, hspec, hspec,
                  aspec, aspec, aspec, aspec,
                  pl.BlockSpec((1, 4 * D), lambda i: (0, 0)),
                  pl.BlockSpec((4 * D, 1), lambda i: (0, 0)),
                  pl.BlockSpec((1, 1), lambda i: (0, 0))],
        out_specs=pl.BlockSpec((BR, 1), lambda i: (i, 0)),
        out_shape=jax.ShapeDtypeStruct((N, 1), jnp.float32),
    )(h1, h2, h3, hp, a1, a2, a3, a4, bl1[None, :], Wl2, bl2[None, :])


def kernel(x, edge_index, edge_attr, batch,
           We1, be1, W1, b1, g1, bt1,
           We2, be2, W2, b2, g2, bt2,
           We3, be3, W3, b3, g3, bt3,
           Wl1, bl1, Wl2, bl2):
    src = edge_index[0].reshape(NW, GCH, GRP, C)
    dst = edge_index[1].reshape(NW, GCH, GRP, C)
    zeros = jnp.zeros((N, D), jnp.float32)

    e1, e2, e3 = _lift(edge_attr, We1, be1, We2, be2, We3, be3)

    agg1 = _gine_sc(x, zeros, e1, src, dst)
    h1 = _dense(agg1, W1, b1, g1, bt1)
    agg2 = _gine_sc(h1, zeros, e2, src, dst)
    h2 = _dense(agg2, W2, b2, g2, bt2)
    agg3 = _gine_sc(h2, zeros, e3, src, dst)
    h3 = _dense(agg3, W3, b3, g3, bt3)

    hp = _pool_sc(h3, zeros, batch)
    return _final(h1, h2, h3, hp, Wl1, bl1, Wl2, bl2)
